# baseline (speedup 1.0000x reference)
import functools

import jax
import jax.numpy as jnp
from jax import lax
from jax.experimental import pallas as pl
from jax.experimental.pallas import tpu as pltpu
from jax.experimental.pallas import tpu_sc as plsc

B = 4096
L = 200
D = 32
NW = 32
BPW = B // NW
NCH = L // 8
TPC = 8 * BPW
SQRT_D = float(D) ** 0.5


def _sc_embed(tok4, seq, emb):
    mesh = plsc.VectorSubcoreMesh(core_axis_name="c", subcore_axis_name="s")

    @functools.partial(
        pl.kernel,
        out_type=[
            jax.ShapeDtypeStruct((L, D // 8, NW, 8, 128), jnp.float32),
            jax.ShapeDtypeStruct((NCH, NW, 8, 128), jnp.float32),
            jax.ShapeDtypeStruct((16,), jnp.int32),
        ],
        mesh=mesh,
        compiler_params=pltpu.CompilerParams(
            needs_layout_passes=False, use_tc_tiling_on_sc=False),
        scratch_types=[
            pltpu.VMEM((8, 128), jnp.int32),
            pltpu.VMEM((8, 128), jnp.int32),
            pltpu.VMEM((TPC, D), jnp.float32),
            pltpu.VMEM((TPC, D), jnp.float32),
            pltpu.VMEM((8, D // 8, 8, 128), jnp.float32),
            pltpu.VMEM((8, 128), jnp.float32),
            pltpu.VMEM((BPW,), jnp.int32),
            pltpu.VMEM((512,), jnp.int32),
            pltpu.VMEM((16,), jnp.int32),
            pltpu.SemaphoreType.DMA,
            pltpu.SemaphoreType.DMA,
            pltpu.SemaphoreType.DMA,
        ],
    )
    def body(tok_hbm, seq_hbm, emb_hbm, x_hbm, mask_hbm, ts_hbm,
             tokv0, tokv1, raw0, raw1, outv, maskv,
             seqlen_v, seqstage_v, ts_v, semg0, semg1, semw):
        wid = lax.axis_index("s") * 2 + lax.axis_index("c")
        iota = lax.iota(jnp.int32, 16)
        c_zero = jnp.zeros((16,), jnp.float32)
        c_one = jnp.full((16,), 1.0, jnp.float32)
        c_scale = jnp.full((16,), SQRT_D, jnp.float32)

        pltpu.sync_copy(seq_hbm.at[pl.ds(wid * BPW, BPW)], seqlen_v)

        @pl.when(wid == 0)
        def _():
            def mbody(i, acc):
                pltpu.sync_copy(seq_hbm.at[pl.ds(i * 512, 512)], seqstage_v)

                def m2(j, a):
                    return jnp.maximum(a, seqstage_v[pl.ds(j * 16, 16)])

                return lax.fori_loop(0, 32, m2, acc)

            acc = lax.fori_loop(0, B // 512, mbody, jnp.zeros((16,), jnp.int32))
            for k in (8, 4, 2, 1):
                ts_v[...] = acc
                acc = jnp.maximum(acc, plsc.load_gather(ts_v, [iota ^ k]))
            ts_v[...] = acc
            pltpu.sync_copy(ts_v, ts_hbm)

        def fire(c, tokv, raw, semg):
            pltpu.sync_copy(tok_hbm.at[c, wid], tokv)
            for j in range(8):
                pltpu.async_copy(emb_hbm.at[tokv.at[j]],
                                 raw.at[pl.ds(j * 128, 128)], semg)

        def drain(tokv, raw, semg):
            for j in range(8):
                pltpu.make_async_copy(emb_hbm.at[tokv.at[j]],
                                      raw.at[pl.ds(j * 128, 128)], semg).wait()

        def compute(lt, raw):
            @plsc.parallel_loop(0, 8, unroll=2)
            def g8(g):
                lenv = seqlen_v[pl.ds(g * 16, 16)]
                base_tok = g * 16 + iota
                for lr in range(8):
                    m = lenv > (lt * 8 + lr)
                    s = jnp.where(m, c_scale, c_zero)
                    maskv[lr, pl.ds(g * 16, 16)] = jnp.where(m, c_one, c_zero)
                    tokidx = base_tok + lr * 128
                    for d in range(D):
                        dv = jnp.full((16,), d, jnp.int32)
                        v = plsc.load_gather(raw, [tokidx, dv])
                        outv[lr, d // 8, d % 8, pl.ds(g * 16, 16)] = v * s

        def fire_wb(lt):
            pltpu.async_copy(outv, x_hbm.at[pl.ds(lt * 8, 8), :, wid, :, :],
                             semw)
            pltpu.async_copy(maskv, mask_hbm.at[lt, wid], semw)

        def drain_wb(lt):
            pltpu.make_async_copy(outv,
                                  x_hbm.at[pl.ds(lt * 8, 8), :, wid, :, :],
                                  semw).wait()
            pltpu.make_async_copy(maskv, mask_hbm.at[lt, wid], semw).wait()

        fire(0, tokv0, raw0, semg0)

        def pipe(cc, _):
            c0 = 2 * cc
            fire(c0 + 1, tokv1, raw1, semg1)
            drain(tokv0, raw0, semg0)

            @pl.when(c0 > 0)
            def _():
                drain_wb(c0)

            compute(c0, raw0)
            fire_wb(c0)

            c1 = 2 * cc + 1
            fire(c1 + 1, tokv0, raw0, semg0)
            drain(tokv1, raw1, semg1)
            drain_wb(c1)
            compute(c1, raw1)
            fire_wb(c1)
            return 0

        lax.fori_loop(0, (NCH - 1) // 2, pipe, 0)
        drain(tokv0, raw0, semg0)
        drain_wb(NCH - 1)
        compute(NCH - 1, raw0)
        fire_wb(NCH - 1)
        drain_wb(NCH - 1)

    return body(tok4, seq, emb)


def kernel(token_ids, seq_lengths, embeddings):
    tok4 = token_ids.T.reshape(NCH, 8, NW, 128).transpose(0, 2, 1, 3)
    x5, mask4, ts = _sc_embed(tok4, seq_lengths, embeddings)
    x = x5.transpose(2, 4, 0, 1, 3).reshape(B, L, D)
    mask = mask4.transpose(1, 3, 0, 2).reshape(B, L)
    return (x, mask, ts[0])


# speedup vs baseline: 1.7109x; 1.7109x over previous
import functools

import jax
import jax.numpy as jnp
from jax import lax
from jax.experimental import pallas as pl
from jax.experimental.pallas import tpu as pltpu
from jax.experimental.pallas import tpu_sc as plsc

B = 4096
L = 200
D = 32
NW = 32
BPW = B // NW
NCH = L // 8
TPC = 8 * BPW
SQRT_D = float(D) ** 0.5


def _sc_embed(tok4, seq, emb):
    mesh = plsc.VectorSubcoreMesh(core_axis_name="c", subcore_axis_name="s")

    @functools.partial(
        pl.kernel,
        out_type=[
            jax.ShapeDtypeStruct((L, (D // 8) * NW * 8 * 128), jnp.float32),
            jax.ShapeDtypeStruct((NCH, NW, 8, 128), jnp.float32),
            jax.ShapeDtypeStruct((16,), jnp.int32),
        ],
        mesh=mesh,
        compiler_params=pltpu.CompilerParams(
            needs_layout_passes=False, use_tc_tiling_on_sc=False),
        scratch_types=[
            pltpu.VMEM((8, 128), jnp.int32),
            pltpu.VMEM((8, 128), jnp.int32),
            pltpu.VMEM((TPC, D), jnp.float32),
            pltpu.VMEM((TPC, D), jnp.float32),
            pltpu.VMEM((8, 1024), jnp.float32),
            pltpu.VMEM((8, 1024), jnp.float32),
            pltpu.VMEM((8, 1024), jnp.float32),
            pltpu.VMEM((8, 1024), jnp.float32),
            pltpu.VMEM((8, 128), jnp.float32),
            pltpu.VMEM((BPW,), jnp.int32),
            pltpu.VMEM((512,), jnp.int32),
            pltpu.VMEM((16,), jnp.int32),
            pltpu.SemaphoreType.DMA,
            pltpu.SemaphoreType.DMA,
            pltpu.SemaphoreType.DMA,
        ],
    )
    def body(tok_hbm, seq_hbm, emb_hbm, x_hbm, mask_hbm, ts_hbm,
             tokv0, tokv1, raw0, raw1, outv0, outv1, outv2, outv3, maskv,
             seqlen_v, seqstage_v, ts_v, semg0, semg1, semw):
        outvs = (outv0, outv1, outv2, outv3)
        wid = lax.axis_index("s") * 2 + lax.axis_index("c")
        iota = lax.iota(jnp.int32, 16)
        c_zero = jnp.zeros((16,), jnp.float32)
        c_one = jnp.full((16,), 1.0, jnp.float32)
        c_scale = jnp.full((16,), SQRT_D, jnp.float32)

        pltpu.sync_copy(seq_hbm.at[pl.ds(wid * BPW, BPW)], seqlen_v)

        @pl.when(wid == 0)
        def _():
            def mbody(i, acc):
                pltpu.sync_copy(seq_hbm.at[pl.ds(i * 512, 512)], seqstage_v)

                def m2(j, a):
                    return jnp.maximum(a, seqstage_v[pl.ds(j * 16, 16)])

                return lax.fori_loop(0, 32, m2, acc)

            acc = lax.fori_loop(0, B // 512, mbody, jnp.zeros((16,), jnp.int32))
            for k in (8, 4, 2, 1):
                ts_v[...] = acc
                acc = jnp.maximum(acc, plsc.load_gather(ts_v, [iota ^ k]))
            ts_v[...] = acc
            pltpu.sync_copy(ts_v, ts_hbm)

        def fire(c, tokv, raw, semg):
            pltpu.sync_copy(tok_hbm.at[c, wid], tokv)
            for j in range(8):
                pltpu.async_copy(emb_hbm.at[tokv.at[j]],
                                 raw.at[pl.ds(j * 128, 128)], semg)

        def drain(tokv, raw, semg):
            for j in range(8):
                pltpu.make_async_copy(emb_hbm.at[tokv.at[j]],
                                      raw.at[pl.ds(j * 128, 128)], semg).wait()

        def compute(lt, raw):
            iota7 = iota & 7
            def g8(g, _):
                lenv = seqlen_v[pl.ds(g * 16, 16)]
                brv = g * 16 + iota
                inner_base = (iota7 << 7) + brv
                for lr in range(8):
                    m = lenv > (lt * 8 + lr)
                    s = jnp.where(m, c_scale, c_zero)
                    maskv[lr, pl.ds(g * 16, 16)] = jnp.where(m, c_one, c_zero)
                    tokidx = brv + lr * 128
                    lr_c = jnp.full((16,), lr, jnp.int32)
                    for d0 in range(0, D, 2):
                        rots = [iota7 ^ (d0 + k) for k in range(2)]
                        vs = [plsc.load_gather(raw, [tokidx, r]) for r in rots]
                        ws = [v * s for v in vs]
                        for k in range(2):
                            d = d0 + k
                            inner = inner_base ^ ((d & 7) << 7)
                            plsc.store_scatter(outvs[d // 8], [lr_c, inner],
                                               ws[k])
                return 0

            lax.fori_loop(0, 8, g8, 0)

        def fire_wb(lt):
            for dt in range(4):
                pltpu.async_copy(
                    outvs[dt],
                    x_hbm.at[pl.ds(lt * 8, 8),
                             pl.ds(dt * 32768 + wid * 1024, 1024)], semw)
            pltpu.async_copy(maskv, mask_hbm.at[lt, wid], semw)

        def drain_wb(lt):
            for dt in range(4):
                pltpu.make_async_copy(
                    outvs[dt],
                    x_hbm.at[pl.ds(lt * 8, 8),
                             pl.ds(dt * 32768 + wid * 1024, 1024)],
                    semw).wait()
            pltpu.make_async_copy(maskv, mask_hbm.at[lt, wid], semw).wait()

        fire(0, tokv0, raw0, semg0)

        def pipe(cc, _):
            c0 = 2 * cc
            fire(c0 + 1, tokv1, raw1, semg1)
            drain(tokv0, raw0, semg0)

            @pl.when(c0 > 0)
            def _():
                drain_wb(c0)

            compute(c0, raw0)
            fire_wb(c0)

            c1 = 2 * cc + 1
            fire(c1 + 1, tokv0, raw0, semg0)
            drain(tokv1, raw1, semg1)
            drain_wb(c1)
            compute(c1, raw1)
            fire_wb(c1)
            return 0

        lax.fori_loop(0, (NCH - 1) // 2, pipe, 0)
        drain(tokv0, raw0, semg0)
        drain_wb(NCH - 1)
        compute(NCH - 1, raw0)
        fire_wb(NCH - 1)
        drain_wb(NCH - 1)

    return body(tok4, seq, emb)


def kernel(token_ids, seq_lengths, embeddings):
    tok4 = token_ids.T.reshape(NCH, 8, NW, 128).transpose(0, 2, 1, 3)
    x2, mask4, ts = _sc_embed(tok4, seq_lengths, embeddings)
    x5 = x2.reshape(L, D // 8, NW, 8, 128)
    x = x5.transpose(2, 4, 0, 1, 3).reshape(B, L, D)
    mask = mask4.transpose(1, 3, 0, 2).reshape(B, L)
    return (x, mask, ts[0])
